# baseline (speedup 1.0000x reference)
import math

import jax
import jax.numpy as jnp
from jax import lax
from jax.experimental import pallas as pl
from jax.experimental.pallas import tpu as pltpu
from jax.experimental.pallas import tpu_sc as plsc

_B, _S, _D = 16, 2048, 1024
_TEMP = 0.07
_P = 64
_BETA = 0.4
_TS = 512
_NST = _S // _TS

_SC_INFO = plsc.get_sparse_core_info()
_NW = _SC_INFO.num_cores * _SC_INFO.num_subcores
_BP = _B * _P
_BPW = _BP // _NW


def _main_body(xt_ref, xr_ref, xh_ref, xi_ref, mask_ref,
               w1t_ref, w1r_ref, b1_ref, wiq_ref, biq_ref, whq_ref, bhq_ref,
               wk_ref, wfc2_ref,
               idx_ref, w_ref, qt_scr, u_scr, scores_scr):
    f32 = jnp.float32
    b = pl.program_id(0)
    st = pl.program_id(1)

    @pl.when(st == 0)
    def _prep():
        img_q = jnp.dot(xi_ref[0], wiq_ref[...], preferred_element_type=f32) + biq_ref[...]
        head_q = jnp.dot(xh_ref[0], whq_ref[...], preferred_element_type=f32) + bhq_ref[...]
        qc = img_q * _BETA + head_q * (1.0 - _BETA)
        scale = 1.0 / (math.sqrt(_D) * _TEMP)
        qt = jax.lax.dot_general(qc, wk_ref[...], (((1,), (1,)), ((), ())),
                                 preferred_element_type=f32) * scale
        qt_scr[...] = qt
        u_scr[...] = jax.lax.dot_general(qt, wfc2_ref[...], (((1,), (1,)), ((), ())),
                                         preferred_element_type=f32)

    hh = _TS // 2
    qt = qt_scr[...]
    u = u_scr[...]
    for half in range(2):
        sl = slice(half * hh, (half + 1) * hh)
        xt = xt_ref[0, sl]
        xr = xr_ref[0, sl]
        z = (jnp.dot(xt, w1t_ref[...], preferred_element_type=f32)
             + jnp.dot(xr, w1r_ref[...], preferred_element_type=f32)
             + b1_ref[...])
        h = jnp.where(z >= 0.0, z, 0.01 * z)
        s = (jnp.sum(xt * qt, axis=1)
             + jnp.sum(h * u, axis=1))
        s = jnp.where(mask_ref[0, 0, sl] < 0.01, -jnp.inf, s)
        scores_scr[pl.ds(b, 1), pl.ds(st * _TS + half * hh, hh)] = s.reshape(1, hh)

    @pl.when(jnp.logical_and(b == _B - 1, st == _NST - 1))
    def _topk():
        sc = scores_scr[...]
        m = jnp.max(sc, axis=1, keepdims=True)
        denom = jnp.sum(jnp.exp(sc - m), axis=1, keepdims=True)
        cur = sc
        col = jax.lax.broadcasted_iota(jnp.int32, (_B, _S), 1)
        lane = jax.lax.broadcasted_iota(jnp.int32, (_B, _P), 1)
        row_off = jax.lax.broadcasted_iota(jnp.int32, (_B, _P), 0) * _S
        idx_acc = jnp.zeros((_B, _P), jnp.int32)
        val_acc = jnp.zeros((_B, _P), f32)
        for i in range(_P):
            mx = jnp.max(cur, axis=1, keepdims=True)
            first = jnp.min(jnp.where(cur == mx, col, _S), axis=1, keepdims=True)
            idx_acc = jnp.where(lane == i, first, idx_acc)
            val_acc = jnp.where(lane == i, jnp.exp(mx - m), val_acc)
            cur = jnp.where(col == first, -jnp.inf, cur)
        idx_ref[...] = idx_acc + row_off
        w_ref[:, 0, :] = val_acc / denom


def _sc_gather_body(xt_hbm, xr_hbm, idx_hbm, outt_hbm, outr_hbm,
                    idx_v, rows_v, sem):
    nc = _SC_INFO.num_cores
    wid = lax.axis_index("s") * nc + lax.axis_index("c")
    base = wid * _BPW
    pltpu.sync_copy(idx_hbm.at[pl.ds(base, _BPW)], idx_v)
    pltpu.async_copy(xt_hbm.at[idx_v], rows_v, sem).wait()
    pltpu.sync_copy(rows_v, outt_hbm.at[pl.ds(base, _BPW)])
    pltpu.async_copy(xr_hbm.at[idx_v], rows_v, sem).wait()
    pltpu.sync_copy(rows_v, outr_hbm.at[pl.ds(base, _BPW)])


def _epi_body(xt_ref, xr_ref, w_ref, wfull_ref, xh_ref, xi_ref,
              w1t_ref, w1r_ref, b1_ref, wfc2_ref, bfc2_ref, wv_ref, bv_ref,
              wproj_ref, bproj_ref, wft_ref, bft_ref, wfi_ref, bfi_ref,
              wftr_ref, bftr_ref, wgt_ref, bgt_ref, wgi_ref, bgi_ref,
              out_ref, t1_scr, t2_scr):
    f32 = jnp.float32
    b = pl.program_id(0)
    xt = xt_ref[0]
    xr = xr_ref[0]
    z = (jnp.dot(xt, w1t_ref[...], preferred_element_type=f32)
         + jnp.dot(xr, w1r_ref[...], preferred_element_type=f32)
         + b1_ref[...])
    h = jnp.where(z >= 0.0, z, 0.01 * z)
    w = w_ref[0]
    t1_scr[pl.ds(b, 1)] = jnp.dot(w, xt, preferred_element_type=f32)
    t2_scr[pl.ds(b, 1)] = jnp.dot(w, h, preferred_element_type=f32)

    @pl.when(b == _B - 1)
    def _finale():
        t1 = t1_scr[...]
        t2 = t2_scr[...]
        sw_col = jnp.sum(wfull_ref[:, 0, :], axis=1, keepdims=True)
        reltail = (t1 + jnp.dot(t2, wfc2_ref[...], preferred_element_type=f32)
                   + sw_col * bfc2_ref[...])
        ctx = jnp.dot(reltail, wv_ref[...], preferred_element_type=f32) + sw_col * bv_ref[...]
        x_triple = jnp.dot(ctx, wproj_ref[...], preferred_element_type=f32) + bproj_ref[...]
        triple_out = jnp.dot(x_triple, wftr_ref[...], preferred_element_type=f32) + bftr_ref[...]
        xh = xh_ref[...]
        xi = xi_ref[...]
        text_out = jnp.dot(xh, wft_ref[...], preferred_element_type=f32) + bft_ref[...]
        img_out = jnp.dot(xi, wfi_ref[...], preferred_element_type=f32) + bfi_ref[...]
        tg = jax.nn.sigmoid(jnp.dot(xh, wgt_ref[...], preferred_element_type=f32) + bgt_ref[...])
        ig = jax.nn.sigmoid(jnp.dot(xi, wgi_ref[...], preferred_element_type=f32) + bgi_ref[...])
        out_ref[...] = triple_out + text_out * tg + img_out * ig


def kernel(x_head, x_rel, x_tail, x_mask, x_img, W_fc1, b_fc1, W_fc2, b_fc2,
           W_k, b_k, W_v, b_v, W_proj, b_proj, W_iq, b_iq, W_hq, b_hq,
           W_ft, b_ft, W_fi, b_fi, W_ftr, b_ftr, W_gt, b_gt, W_gi, b_gi):
    f32 = jnp.float32
    r2 = lambda b: b.reshape(1, _D)
    w1t = W_fc1[:_D]
    w1r = W_fc1[_D:]
    xh3 = x_head.reshape(_B, 1, _D)
    xi3 = x_img.reshape(_B, 1, _D)
    mask3 = x_mask.reshape(_B, 1, _S)

    full2 = lambda a, c: pl.BlockSpec((a, c), lambda b, s: (0, 0))

    idx, w = pl.pallas_call(
        _main_body,
        grid=(_B, _NST),
        in_specs=[
            pl.BlockSpec((1, _TS, _D), lambda b, s: (b, s, 0)),
            pl.BlockSpec((1, _TS, _D), lambda b, s: (b, s, 0)),
            pl.BlockSpec((1, 1, _D), lambda b, s: (b, 0, 0)),
            pl.BlockSpec((1, 1, _D), lambda b, s: (b, 0, 0)),
            pl.BlockSpec((1, 1, _TS), lambda b, s: (b, 0, s)),
            full2(_D, _D), full2(_D, _D), full2(1, _D),
            full2(_D, _D), full2(1, _D), full2(_D, _D), full2(1, _D),
            full2(_D, _D), full2(_D, _D),
        ],
        out_specs=[
            pl.BlockSpec((_B, _P), lambda b, s: (0, 0)),
            pl.BlockSpec((_B, 1, _P), lambda b, s: (0, 0, 0)),
        ],
        out_shape=[
            jax.ShapeDtypeStruct((_B, _P), jnp.int32),
            jax.ShapeDtypeStruct((_B, 1, _P), f32),
        ],
        scratch_shapes=[
            pltpu.VMEM((1, _D), f32),
            pltpu.VMEM((1, _D), f32),
            pltpu.VMEM((_B, _S), f32),
        ],
    )(x_tail, x_rel, xh3, xi3, mask3,
      w1t, w1r, r2(b_fc1), W_iq, r2(b_iq), W_hq, r2(b_hq), W_k, W_fc2)

    xt_flat = x_tail.reshape(_B * _S, _D)
    xr_flat = x_rel.reshape(_B * _S, _D)
    gidx = idx.reshape(_BP)
    sc_gather = pl.kernel(
        _sc_gather_body,
        mesh=plsc.VectorSubcoreMesh(core_axis_name="c", subcore_axis_name="s"),
        out_type=[jax.ShapeDtypeStruct((_BP, _D), f32)] * 2,
        scratch_types=[
            pltpu.VMEM((_BPW,), jnp.int32),
            pltpu.VMEM((_BPW, _D), f32),
            pltpu.SemaphoreType.DMA,
        ],
    )
    xt_top, xr_top = sc_gather(xt_flat, xr_flat, gidx)

    out = pl.pallas_call(
        _epi_body,
        grid=(_B,),
        in_specs=[
            pl.BlockSpec((1, _P, _D), lambda b: (b, 0, 0)),
            pl.BlockSpec((1, _P, _D), lambda b: (b, 0, 0)),
            pl.BlockSpec((1, 1, _P), lambda b: (b, 0, 0)),
            pl.BlockSpec((_B, 1, _P), lambda b: (0, 0, 0)),
            pl.BlockSpec((_B, _D), lambda b: (0, 0)),
            pl.BlockSpec((_B, _D), lambda b: (0, 0)),
            pl.BlockSpec((_D, _D), lambda b: (0, 0)),
            pl.BlockSpec((_D, _D), lambda b: (0, 0)),
            pl.BlockSpec((1, _D), lambda b: (0, 0)),
            pl.BlockSpec((_D, _D), lambda b: (0, 0)),
            pl.BlockSpec((1, _D), lambda b: (0, 0)),
            pl.BlockSpec((_D, _D), lambda b: (0, 0)),
            pl.BlockSpec((1, _D), lambda b: (0, 0)),
            pl.BlockSpec((_D, _D), lambda b: (0, 0)),
            pl.BlockSpec((1, _D), lambda b: (0, 0)),
            pl.BlockSpec((_D, _D), lambda b: (0, 0)),
            pl.BlockSpec((1, _D), lambda b: (0, 0)),
            pl.BlockSpec((_D, _D), lambda b: (0, 0)),
            pl.BlockSpec((1, _D), lambda b: (0, 0)),
            pl.BlockSpec((_D, _D), lambda b: (0, 0)),
            pl.BlockSpec((1, _D), lambda b: (0, 0)),
            pl.BlockSpec((_D, _D), lambda b: (0, 0)),
            pl.BlockSpec((1, _D), lambda b: (0, 0)),
            pl.BlockSpec((_D, _D), lambda b: (0, 0)),
            pl.BlockSpec((1, _D), lambda b: (0, 0)),
        ],
        out_specs=pl.BlockSpec((_B, _D), lambda b: (0, 0)),
        out_shape=jax.ShapeDtypeStruct((_B, _D), f32),
        scratch_shapes=[
            pltpu.VMEM((_B, _D), f32),
            pltpu.VMEM((_B, _D), f32),
        ],
    )(xt_top.reshape(_B, _P, _D), xr_top.reshape(_B, _P, _D), w, w,
      x_head, x_img,
      w1t, w1r, r2(b_fc1), W_fc2, r2(b_fc2), W_v, r2(b_v),
      W_proj, r2(b_proj), W_ft, r2(b_ft), W_fi, r2(b_fi),
      W_ftr, r2(b_ftr), W_gt, r2(b_gt), W_gi, r2(b_gi))
    return out


# speedup vs baseline: 1.0920x; 1.0920x over previous
import math

import jax
import jax.numpy as jnp
from jax import lax
from jax.experimental import pallas as pl
from jax.experimental.pallas import tpu as pltpu
from jax.experimental.pallas import tpu_sc as plsc

_B, _S, _D = 16, 2048, 1024
_TEMP = 0.07
_P = 64
_BETA = 0.4
_TS = 1024
_NST = _S // _TS

_SC_INFO = plsc.get_sparse_core_info()
_NW = _SC_INFO.num_cores * _SC_INFO.num_subcores
_BP = _B * _P
_BPW = _BP // _NW


def _main_body(xt_ref, xr_ref, xh_ref, xi_ref, maskt_ref,
               w1t_ref, w1r_ref, b1_ref, wiq_ref, biq_ref, whq_ref, bhq_ref,
               wk_ref, wfc2_ref,
               idx_ref, w_ref, qtp_scr, up_scr, scores_scr):
    f32 = jnp.float32
    bf16 = jnp.bfloat16
    b = pl.program_id(0)
    st = pl.program_id(1)

    @pl.when(jnp.logical_and(b == 0, st == 0))
    def _init():
        scores_scr[...] = jnp.full((_S, 128), -jnp.inf, f32)

    @pl.when(st == 0)
    def _prep():
        img_q = jnp.dot(xi_ref[0], wiq_ref[...], preferred_element_type=f32) + biq_ref[...]
        head_q = jnp.dot(xh_ref[0], whq_ref[...], preferred_element_type=f32) + bhq_ref[...]
        qc = img_q * _BETA + head_q * (1.0 - _BETA)
        scale = 1.0 / (math.sqrt(_D) * _TEMP)
        qt_row = jax.lax.dot_general(qc, wk_ref[...], (((1,), (1,)), ((), ())),
                                     preferred_element_type=f32) * scale
        u_row = jax.lax.dot_general(qt_row, wfc2_ref[...], (((1,), (1,)), ((), ())),
                                    preferred_element_type=f32)
        qtp_scr[...] = qt_row
        up_scr[...] = u_row

    hh = _TS // 2
    lane128 = jax.lax.broadcasted_iota(jnp.int32, (hh, 128), 1)
    qt = qtp_scr[...]
    u = up_scr[...]
    for half in range(2):
        sl = slice(half * hh, (half + 1) * hh)
        xt_f = xt_ref[0, sl]
        xr = xr_ref[0, sl].astype(bf16)
        z = (jnp.dot(xt_f.astype(bf16), w1t_ref[...], preferred_element_type=f32)
             + jnp.dot(xr, w1r_ref[...], preferred_element_type=f32)
             + b1_ref[...])
        h = jnp.where(z >= 0.0, z, 0.01 * z)
        s_col = (jnp.sum(xt_f * qt, axis=1, keepdims=True)
                 + jnp.sum(h * u, axis=1, keepdims=True))
        rows = pl.ds(st * _TS + half * hh, hh)
        scores_scr[rows, :] = jnp.where(lane128 == b,
                                        jnp.broadcast_to(s_col, (hh, 128)),
                                        scores_scr[rows, :])

    @pl.when(jnp.logical_and(b == _B - 1, st == _NST - 1))
    def _topk():
        sc_cols = scores_scr[...]
        sc_cols = jnp.where(maskt_ref[...] < 0.01, -jnp.inf, sc_cols)
        sc = jnp.transpose(sc_cols)[:_B]
        m = jnp.max(sc, axis=1, keepdims=True)
        denom = jnp.sum(jnp.exp(sc - m), axis=1, keepdims=True)
        cur = sc
        col = jax.lax.broadcasted_iota(jnp.int32, (_B, _S), 1)
        lane = jax.lax.broadcasted_iota(jnp.int32, (_B, _P), 1)
        row_off = jax.lax.broadcasted_iota(jnp.int32, (_B, _P), 0) * _S
        idx_acc = jnp.zeros((_B, _P), jnp.int32)
        val_acc = jnp.zeros((_B, _P), f32)
        for i in range(_P):
            mx = jnp.max(cur, axis=1, keepdims=True)
            first = jnp.min(jnp.where(cur == mx, col, _S), axis=1, keepdims=True)
            idx_acc = jnp.where(lane == i, first, idx_acc)
            val_acc = jnp.where(lane == i, jnp.exp(mx - m), val_acc)
            cur = jnp.where(col == first, -jnp.inf, cur)
        idx_ref[...] = idx_acc + row_off
        w_ref[:, 0, :] = val_acc / denom


def _sc_gather_body(xt_hbm, xr_hbm, idx_hbm, outt_hbm, outr_hbm,
                    idx_v, rows_t, rows_r, sem_t, sem_r):
    nc = _SC_INFO.num_cores
    wid = lax.axis_index("s") * nc + lax.axis_index("c")
    base = wid * _BPW
    pltpu.sync_copy(idx_hbm.at[pl.ds(base, _BPW)], idx_v)
    ct = pltpu.async_copy(xt_hbm.at[idx_v], rows_t, sem_t)
    cr = pltpu.async_copy(xr_hbm.at[idx_v], rows_r, sem_r)
    ct.wait()
    pltpu.sync_copy(rows_t, outt_hbm.at[pl.ds(base, _BPW)])
    cr.wait()
    pltpu.sync_copy(rows_r, outr_hbm.at[pl.ds(base, _BPW)])


def _epi_body(xt_ref, xr_ref, w_ref, wfull_ref, xh_ref, xi_ref,
              w1t_ref, w1r_ref, b1_ref, wfc2_ref, bfc2_ref, wv_ref, bv_ref,
              wproj_ref, bproj_ref, wft_ref, bft_ref, wfi_ref, bfi_ref,
              wftr_ref, bftr_ref, wgt_ref, bgt_ref, wgi_ref, bgi_ref,
              out_ref, t1_scr, t2_scr):
    f32 = jnp.float32
    b = pl.program_id(0)
    xt = xt_ref[0]
    xr = xr_ref[0]
    z = (jnp.dot(xt, w1t_ref[...], preferred_element_type=f32)
         + jnp.dot(xr, w1r_ref[...], preferred_element_type=f32)
         + b1_ref[...])
    h = jnp.where(z >= 0.0, z, 0.01 * z)
    w = w_ref[0]
    t1_scr[pl.ds(b, 1)] = jnp.dot(w, xt, preferred_element_type=f32)
    t2_scr[pl.ds(b, 1)] = jnp.dot(w, h, preferred_element_type=f32)

    @pl.when(b == _B - 1)
    def _finale():
        t1 = t1_scr[...]
        t2 = t2_scr[...]
        sw_col = jnp.sum(wfull_ref[:, 0, :], axis=1, keepdims=True)
        reltail = (t1 + jnp.dot(t2, wfc2_ref[...], preferred_element_type=f32)
                   + sw_col * bfc2_ref[...])
        ctx = jnp.dot(reltail, wv_ref[...], preferred_element_type=f32) + sw_col * bv_ref[...]
        x_triple = jnp.dot(ctx, wproj_ref[...], preferred_element_type=f32) + bproj_ref[...]
        triple_out = jnp.dot(x_triple, wftr_ref[...], preferred_element_type=f32) + bftr_ref[...]
        xh = xh_ref[...]
        xi = xi_ref[...]
        text_out = jnp.dot(xh, wft_ref[...], preferred_element_type=f32) + bft_ref[...]
        img_out = jnp.dot(xi, wfi_ref[...], preferred_element_type=f32) + bfi_ref[...]
        tg = jax.nn.sigmoid(jnp.dot(xh, wgt_ref[...], preferred_element_type=f32) + bgt_ref[...])
        ig = jax.nn.sigmoid(jnp.dot(xi, wgi_ref[...], preferred_element_type=f32) + bgi_ref[...])
        out_ref[...] = triple_out + text_out * tg + img_out * ig


def kernel(x_head, x_rel, x_tail, x_mask, x_img, W_fc1, b_fc1, W_fc2, b_fc2,
           W_k, b_k, W_v, b_v, W_proj, b_proj, W_iq, b_iq, W_hq, b_hq,
           W_ft, b_ft, W_fi, b_fi, W_ftr, b_ftr, W_gt, b_gt, W_gi, b_gi):
    f32 = jnp.float32
    bf16 = jnp.bfloat16
    r2 = lambda b: b.reshape(1, _D)
    w1t = W_fc1[:_D]
    w1r = W_fc1[_D:]
    w1t_b = w1t.astype(bf16)
    w1r_b = w1r.astype(bf16)
    xh3 = x_head.reshape(_B, 1, _D)
    xi3 = x_img.reshape(_B, 1, _D)
    maskt = jnp.pad(x_mask.T, ((0, 0), (0, 128 - _B)), constant_values=1.0)

    full2 = lambda a, c: pl.BlockSpec((a, c), lambda b, s: (0, 0))

    idx, w = pl.pallas_call(
        _main_body,
        grid=(_B, _NST),
        in_specs=[
            pl.BlockSpec((1, _TS, _D), lambda b, s: (b, s, 0)),
            pl.BlockSpec((1, _TS, _D), lambda b, s: (b, s, 0)),
            pl.BlockSpec((1, 1, _D), lambda b, s: (b, 0, 0)),
            pl.BlockSpec((1, 1, _D), lambda b, s: (b, 0, 0)),
            full2(_S, 128),
            full2(_D, _D), full2(_D, _D), full2(1, _D),
            full2(_D, _D), full2(1, _D), full2(_D, _D), full2(1, _D),
            full2(_D, _D), full2(_D, _D),
        ],
        out_specs=[
            pl.BlockSpec((_B, _P), lambda b, s: (0, 0)),
            pl.BlockSpec((_B, 1, _P), lambda b, s: (0, 0, 0)),
        ],
        out_shape=[
            jax.ShapeDtypeStruct((_B, _P), jnp.int32),
            jax.ShapeDtypeStruct((_B, 1, _P), f32),
        ],
        scratch_shapes=[
            pltpu.VMEM((1, _D), f32),
            pltpu.VMEM((1, _D), f32),
            pltpu.VMEM((_S, 128), f32),
        ],
    )(x_tail, x_rel, xh3, xi3, maskt,
      w1t_b, w1r_b, r2(b_fc1), W_iq, r2(b_iq), W_hq, r2(b_hq), W_k, W_fc2)

    xt_flat = x_tail.reshape(_B * _S, _D)
    xr_flat = x_rel.reshape(_B * _S, _D)
    gidx = idx.reshape(_BP)
    sc_gather = pl.kernel(
        _sc_gather_body,
        mesh=plsc.VectorSubcoreMesh(core_axis_name="c", subcore_axis_name="s"),
        out_type=[jax.ShapeDtypeStruct((_BP, _D), f32)] * 2,
        scratch_types=[
            pltpu.VMEM((_BPW,), jnp.int32),
            pltpu.VMEM((_BPW, _D), f32),
            pltpu.VMEM((_BPW, _D), f32),
            pltpu.SemaphoreType.DMA,
            pltpu.SemaphoreType.DMA,
        ],
    )
    xt_top, xr_top = sc_gather(xt_flat, xr_flat, gidx)

    out = pl.pallas_call(
        _epi_body,
        grid=(_B,),
        in_specs=[
            pl.BlockSpec((1, _P, _D), lambda b: (b, 0, 0)),
            pl.BlockSpec((1, _P, _D), lambda b: (b, 0, 0)),
            pl.BlockSpec((1, 1, _P), lambda b: (b, 0, 0)),
            pl.BlockSpec((_B, 1, _P), lambda b: (0, 0, 0)),
            pl.BlockSpec((_B, _D), lambda b: (0, 0)),
            pl.BlockSpec((_B, _D), lambda b: (0, 0)),
            pl.BlockSpec((_D, _D), lambda b: (0, 0)),
            pl.BlockSpec((_D, _D), lambda b: (0, 0)),
            pl.BlockSpec((1, _D), lambda b: (0, 0)),
            pl.BlockSpec((_D, _D), lambda b: (0, 0)),
            pl.BlockSpec((1, _D), lambda b: (0, 0)),
            pl.BlockSpec((_D, _D), lambda b: (0, 0)),
            pl.BlockSpec((1, _D), lambda b: (0, 0)),
            pl.BlockSpec((_D, _D), lambda b: (0, 0)),
            pl.BlockSpec((1, _D), lambda b: (0, 0)),
            pl.BlockSpec((_D, _D), lambda b: (0, 0)),
            pl.BlockSpec((1, _D), lambda b: (0, 0)),
            pl.BlockSpec((_D, _D), lambda b: (0, 0)),
            pl.BlockSpec((1, _D), lambda b: (0, 0)),
            pl.BlockSpec((_D, _D), lambda b: (0, 0)),
            pl.BlockSpec((1, _D), lambda b: (0, 0)),
            pl.BlockSpec((_D, _D), lambda b: (0, 0)),
            pl.BlockSpec((1, _D), lambda b: (0, 0)),
            pl.BlockSpec((_D, _D), lambda b: (0, 0)),
            pl.BlockSpec((1, _D), lambda b: (0, 0)),
        ],
        out_specs=pl.BlockSpec((_B, _D), lambda b: (0, 0)),
        out_shape=jax.ShapeDtypeStruct((_B, _D), f32),
        scratch_shapes=[
            pltpu.VMEM((_B, _D), f32),
            pltpu.VMEM((_B, _D), f32),
        ],
    )(xt_top.reshape(_B, _P, _D), xr_top.reshape(_B, _P, _D), w, w,
      x_head, x_img,
      w1t, w1r, r2(b_fc1), W_fc2, r2(b_fc2), W_v, r2(b_v),
      W_proj, r2(b_proj), W_ft, r2(b_ft), W_fi, r2(b_fi),
      W_ftr, r2(b_ftr), W_gt, r2(b_gt), W_gi, r2(b_gi))
    return out
